# Initial kernel scaffold; baseline (speedup 1.0000x reference)
import jax
import jax.numpy as jnp
from jax.experimental import pallas as pl


def kernel(x, edge_index, Wl1, bl1, Wr1, Wl2, bl2, Wr2, Wl3, bl3, Wr3, Wl4, bl4, Wr4):
    raise NotImplementedError("write your pallas kernel here")


# speedup vs baseline: 2.8234x; 2.8234x over previous
import functools

import jax
import jax.numpy as jnp
from jax import lax
from jax.experimental import pallas as pl
from jax.experimental.pallas import tpu as pltpu
from jax.experimental.pallas import tpu_sc as plsc

N = 10000
D = 128
E = 320000

NC = 2
NS = 16
NW = NC * NS
CH = 128
G = 16
K = -(-E // (NW * CH * G)) * G
E_PAD = NW * K * CH
N_PAD = 10240
R = N_PAD // NS
RC = R // CH


@functools.cache
def _make_sc_agg(with_gather: bool):
    mesh = plsc.VectorSubcoreMesh(core_axis_name="c", subcore_axis_name="s",
                                  num_cores=NC, num_subcores=NS)
    scratch = [
        pltpu.VMEM((G, CH), jnp.int32),
        pltpu.VMEM((CH, D), jnp.float32),
        pltpu.VMEM_SHARED((N_PAD, D), jnp.float32),
    ]
    if with_gather:
        scratch += [pltpu.VMEM((G, CH), jnp.int32),
                    pltpu.SemaphoreType.DMA]

    def body(*refs):
        if with_gather:
            (x_hbm, src_hbm, dst_hbm, zrows_hbm,
             out_hbm, dst_v, rows_v, acc, src_v, sem) = refs
        else:
            (dst_hbm, zrows_hbm, ones_hbm,
             out_hbm, dst_v, rows_v, acc) = refs
        c = lax.axis_index("c")
        s = lax.axis_index("s")
        wid = c * NS + s
        base = s * R
        pltpu.sync_copy(zrows_hbm, rows_v)
        for t in range(RC):
            pltpu.sync_copy(rows_v, acc.at[pl.ds(base + t * CH, CH)])
        if not with_gather:
            pltpu.sync_copy(ones_hbm, rows_v)
        plsc.subcore_barrier()

        def outer(jj, carry):
            pltpu.sync_copy(dst_hbm.at[wid, pl.ds(jj * G, G)], dst_v)
            if with_gather:
                pltpu.sync_copy(src_hbm.at[wid, pl.ds(jj * G, G)], src_v)

            def step(g, carry2):
                if with_gather:
                    pltpu.async_copy(x_hbm.at[src_v.at[g]], rows_v,
                                     sem).wait()
                pltpu.sync_copy(rows_v, acc.at[dst_v.at[g]], add=True)
                return carry2

            return lax.fori_loop(0, G, step, carry)

        lax.fori_loop(0, K // G, outer, 0)
        plsc.subcore_barrier()
        for t in range(RC):
            pltpu.sync_copy(acc.at[pl.ds(base + t * CH, CH)], rows_v)
            pltpu.sync_copy(rows_v, out_hbm.at[c, pl.ds(base + t * CH, CH)])

    return pl.kernel(body,
                     out_type=jax.ShapeDtypeStruct((NC, N_PAD, D),
                                                   jnp.float32),
                     mesh=mesh, scratch_types=scratch)


_BR = 1024


def _inv_body(c_ref, o_ref):
    o_ref[...] = 1.0 / jnp.maximum(c_ref[0] + c_ref[1], 1.0)


def _inv_counts(cparts):
    return pl.pallas_call(
        _inv_body,
        grid=(N_PAD // _BR,),
        in_specs=[pl.BlockSpec((NC, _BR, D), lambda i: (0, i, 0))],
        out_specs=pl.BlockSpec((_BR, D), lambda i: (i, 0)),
        out_shape=jax.ShapeDtypeStruct((N_PAD, D), jnp.float32),
    )(cparts)


def _dense_body(p_ref, inv_ref, h_ref, wl_ref, bl_ref, wr_ref, o_ref):
    agg = (p_ref[0] + p_ref[1]) * inv_ref[...]
    y = jnp.dot(agg, wl_ref[...], preferred_element_type=jnp.float32)
    y = y + bl_ref[...] + jnp.dot(h_ref[...], wr_ref[...],
                                  preferred_element_type=jnp.float32)
    o_ref[...] = jnp.where(y >= 0, y, 0.01 * y)


def _dense(p, inv, h, wlT, bl2d, wrT):
    return pl.pallas_call(
        _dense_body,
        grid=(N_PAD // _BR,),
        in_specs=[
            pl.BlockSpec((NC, _BR, D), lambda i: (0, i, 0)),
            pl.BlockSpec((_BR, D), lambda i: (i, 0)),
            pl.BlockSpec((_BR, D), lambda i: (i, 0)),
            pl.BlockSpec((D, D), lambda i: (0, 0)),
            pl.BlockSpec((1, D), lambda i: (0, 0)),
            pl.BlockSpec((D, D), lambda i: (0, 0)),
        ],
        out_specs=pl.BlockSpec((_BR, D), lambda i: (i, 0)),
        out_shape=jax.ShapeDtypeStruct((N_PAD, D), jnp.float32),
    )(p, inv, h, wlT, bl2d, wrT)


def kernel(x, edge_index, Wl1, bl1, Wr1, Wl2, bl2, Wr2,
           Wl3, bl3, Wr3, Wl4, bl4, Wr4):
    src = edge_index[0]
    dst = edge_index[1]
    pad = E_PAD - E
    src_p = jnp.concatenate(
        [src, jnp.zeros((pad,), jnp.int32)]).reshape(NW, K, CH)
    dst_p = jnp.concatenate(
        [dst, jnp.full((pad,), N, jnp.int32)]).reshape(NW, K, CH)
    zrows = jnp.zeros((CH, D), jnp.float32)
    ones_rows = jnp.ones((CH, D), jnp.float32)

    cparts = _make_sc_agg(False)(dst_p, zrows, ones_rows)
    inv = _inv_counts(cparts)

    h = jnp.pad(x, ((0, N_PAD - N), (0, 0)))
    for Wl, bl, Wr in ((Wl1, bl1, Wr1), (Wl2, bl2, Wr2),
                       (Wl3, bl3, Wr3), (Wl4, bl4, Wr4)):
        parts = _make_sc_agg(True)(h, src_p, dst_p, zrows)
        h = _dense(parts, inv, h, Wl.T, bl.reshape(1, D), Wr.T)
    return h[:N]


# speedup vs baseline: 3.0725x; 1.0882x over previous
import functools

import jax
import jax.numpy as jnp
from jax import lax
from jax.experimental import pallas as pl
from jax.experimental.pallas import tpu as pltpu
from jax.experimental.pallas import tpu_sc as plsc

N = 10000
D = 128
E = 320000

NC = 2
NS = 16
NW = NC * NS
CH = 128
G = 16
K = -(-E // (NW * CH * G)) * G
E_PAD = NW * K * CH
N_PAD = 10240
R = N_PAD // NS
RC = R // CH


def _mesh():
    return plsc.VectorSubcoreMesh(core_axis_name="c", subcore_axis_name="s",
                                  num_cores=NC, num_subcores=NS)


def _tile_ids():
    c = lax.axis_index("c")
    s = lax.axis_index("s")
    return c, c * NS + s, s * R


def _zero_acc(zrows_hbm, stage_v, acc, base):
    pltpu.sync_copy(zrows_hbm, stage_v)
    for t in range(RC):
        pltpu.sync_copy(stage_v, acc.at[pl.ds(base + t * CH, CH)])


def _copy_out(acc, stage_v, out_hbm, c, base):
    for t in range(RC):
        pltpu.sync_copy(acc.at[pl.ds(base + t * CH, CH)], stage_v)
        pltpu.sync_copy(stage_v, out_hbm.at[c, pl.ds(base + t * CH, CH)])


@functools.cache
def _make_sc_counts():
    scratch = [
        pltpu.VMEM((G, CH), jnp.int32),
        pltpu.VMEM((CH, D), jnp.float32),
        pltpu.VMEM_SHARED((N_PAD, D), jnp.float32),
    ]

    def body(dst_hbm, zrows_hbm, ones_hbm, out_hbm, dst_v, rows_v, acc):
        c, wid, base = _tile_ids()
        _zero_acc(zrows_hbm, rows_v, acc, base)
        pltpu.sync_copy(ones_hbm, rows_v)
        plsc.subcore_barrier()

        def outer(jj, carry):
            pltpu.sync_copy(dst_hbm.at[wid, pl.ds(jj * G, G)], dst_v)

            def step(g, carry2):
                pltpu.sync_copy(rows_v, acc.at[dst_v.at[g]], add=True)
                return carry2

            return lax.fori_loop(0, G, step, carry)

        lax.fori_loop(0, K // G, outer, 0)
        plsc.subcore_barrier()
        _copy_out(acc, rows_v, out_hbm, c, base)

    return pl.kernel(body,
                     out_type=jax.ShapeDtypeStruct((NC, N_PAD, D),
                                                   jnp.float32),
                     mesh=_mesh(), scratch_types=scratch)


H = K // 2


@functools.cache
def _make_sc_agg():
    scratch = [
        pltpu.VMEM((H, CH), jnp.int32),
        pltpu.VMEM((H, CH), jnp.int32),
        pltpu.VMEM((CH, D), jnp.float32),
        pltpu.VMEM((CH, D), jnp.float32),
        pltpu.VMEM_SHARED((N_PAD, D), jnp.float32),
        pltpu.SemaphoreType.DMA,
        pltpu.SemaphoreType.DMA,
    ]

    def body(x_hbm, src_hbm, dst_hbm, zrows_hbm, out_hbm,
             src_v, dst_v, rows0, rows1, acc, sem0, sem1):
        c, wid, base = _tile_ids()
        _zero_acc(zrows_hbm, rows0, acc, base)
        plsc.subcore_barrier()

        for hh in range(2):
            pltpu.sync_copy(src_hbm.at[wid, pl.ds(hh * H, H)], src_v)
            pltpu.sync_copy(dst_hbm.at[wid, pl.ds(hh * H, H)], dst_v)
            pltpu.async_copy(x_hbm.at[src_v.at[0]], rows0, sem0)

            def pair(jj, carry):
                l0 = jj * 2
                pltpu.make_async_copy(x_hbm.at[src_v.at[l0]], rows0,
                                      sem0).wait()
                pltpu.async_copy(x_hbm.at[src_v.at[l0 + 1]], rows1, sem1)
                pltpu.sync_copy(rows0, acc.at[dst_v.at[l0]], add=True)
                pltpu.make_async_copy(x_hbm.at[src_v.at[l0 + 1]], rows1,
                                      sem1).wait()

                @pl.when(l0 + 2 < H)
                def _():
                    pltpu.async_copy(x_hbm.at[src_v.at[l0 + 2]], rows0,
                                     sem0)

                pltpu.sync_copy(rows1, acc.at[dst_v.at[l0 + 1]], add=True)
                return carry

            lax.fori_loop(0, H // 2, pair, 0)

        plsc.subcore_barrier()
        _copy_out(acc, rows0, out_hbm, c, base)

    return pl.kernel(body,
                     out_type=jax.ShapeDtypeStruct((NC, N_PAD, D),
                                                   jnp.float32),
                     mesh=_mesh(), scratch_types=scratch)


_BR = 1024


def _inv_body(c_ref, o_ref):
    o_ref[...] = 1.0 / jnp.maximum(c_ref[0] + c_ref[1], 1.0)


def _inv_counts(cparts):
    return pl.pallas_call(
        _inv_body,
        grid=(N_PAD // _BR,),
        in_specs=[pl.BlockSpec((NC, _BR, D), lambda i: (0, i, 0))],
        out_specs=pl.BlockSpec((_BR, D), lambda i: (i, 0)),
        out_shape=jax.ShapeDtypeStruct((N_PAD, D), jnp.float32),
    )(cparts)


def _dense_body(p_ref, inv_ref, h_ref, wl_ref, bl_ref, wr_ref, o_ref):
    agg = (p_ref[0] + p_ref[1]) * inv_ref[...]
    y = jnp.dot(agg, wl_ref[...], preferred_element_type=jnp.float32)
    y = y + bl_ref[...] + jnp.dot(h_ref[...], wr_ref[...],
                                  preferred_element_type=jnp.float32)
    o_ref[...] = jnp.where(y >= 0, y, 0.01 * y)


def _dense(p, inv, h, wlT, bl2d, wrT):
    return pl.pallas_call(
        _dense_body,
        grid=(N_PAD // _BR,),
        in_specs=[
            pl.BlockSpec((NC, _BR, D), lambda i: (0, i, 0)),
            pl.BlockSpec((_BR, D), lambda i: (i, 0)),
            pl.BlockSpec((_BR, D), lambda i: (i, 0)),
            pl.BlockSpec((D, D), lambda i: (0, 0)),
            pl.BlockSpec((1, D), lambda i: (0, 0)),
            pl.BlockSpec((D, D), lambda i: (0, 0)),
        ],
        out_specs=pl.BlockSpec((_BR, D), lambda i: (i, 0)),
        out_shape=jax.ShapeDtypeStruct((N_PAD, D), jnp.float32),
    )(p, inv, h, wlT, bl2d, wrT)


def kernel(x, edge_index, Wl1, bl1, Wr1, Wl2, bl2, Wr2,
           Wl3, bl3, Wr3, Wl4, bl4, Wr4):
    src = edge_index[0]
    dst = edge_index[1]
    pad = E_PAD - E
    src_p = jnp.concatenate(
        [src, jnp.zeros((pad,), jnp.int32)]).reshape(NW, K, CH)
    dst_p = jnp.concatenate(
        [dst, jnp.full((pad,), N, jnp.int32)]).reshape(NW, K, CH)
    zrows = jnp.zeros((CH, D), jnp.float32)
    ones_rows = jnp.ones((CH, D), jnp.float32)

    cparts = _make_sc_counts()(dst_p, zrows, ones_rows)
    inv = _inv_counts(cparts)

    h = jnp.pad(x, ((0, N_PAD - N), (0, 0)))
    for Wl, bl, Wr in ((Wl1, bl1, Wr1), (Wl2, bl2, Wr2),
                       (Wl3, bl3, Wr3), (Wl4, bl4, Wr4)):
        parts = _make_sc_agg()(h, src_p, dst_p, zrows)
        h = _dense(parts, inv, h, Wl.T, bl.reshape(1, D), Wr.T)
    return h[:N]


# speedup vs baseline: 3.7477x; 1.2197x over previous
import functools

import jax
import jax.numpy as jnp
from jax import lax
from jax.experimental import pallas as pl
from jax.experimental.pallas import tpu as pltpu
from jax.experimental.pallas import tpu_sc as plsc

N = 10000
D = 128
E = 320000

NC = 2
NS = 16
CH = 128
K_ALL = -(-E // (NS * CH * 32)) * 32
E_PAD = NS * K_ALL * CH
S = 32
K_BIG = 128
BIG_CORE = 0
N_PAD = 10240
R = N_PAD // NS
RC = R // CH


def _mesh():
    return plsc.VectorSubcoreMesh(core_axis_name="c", subcore_axis_name="s",
                                  num_cores=NC, num_subcores=NS)


def _tile_ids():
    c = lax.axis_index("c")
    s = lax.axis_index("s")
    return c, s, s * R


def _zero_acc(zrows_hbm, stage_v, acc, base):
    pltpu.sync_copy(zrows_hbm, stage_v)
    for t in range(RC):
        pltpu.sync_copy(stage_v, acc.at[pl.ds(base + t * CH, CH)])


def _copy_out(acc, stage_v, out_hbm, c, base):
    for t in range(RC):
        pltpu.sync_copy(acc.at[pl.ds(base + t * CH, CH)], stage_v)
        pltpu.sync_copy(stage_v, out_hbm.at[c, pl.ds(base + t * CH, CH)])


@functools.cache
def _make_sc_counts():
    scratch = [
        pltpu.VMEM((S, CH), jnp.int32),
        pltpu.VMEM((CH, D), jnp.float32),
        pltpu.VMEM_SHARED((N_PAD, D), jnp.float32),
    ]
    HALF = K_ALL // NC

    def body(dst_hbm, zrows_hbm, ones_hbm, out_hbm, dst_v, rows_v, acc):
        c, s, base = _tile_ids()
        _zero_acc(zrows_hbm, rows_v, acc, base)
        pltpu.sync_copy(ones_hbm, rows_v)
        plsc.subcore_barrier()
        first = c * HALF

        def outer(q, carry):
            pltpu.sync_copy(dst_hbm.at[s, pl.ds(first + q * S, S)], dst_v)

            def step(g, carry2):
                pltpu.sync_copy(rows_v, acc.at[dst_v.at[g]], add=True)
                return carry2

            return lax.fori_loop(0, S, step, carry)

        lax.fori_loop(0, HALF // S, outer, 0)
        plsc.subcore_barrier()
        _copy_out(acc, rows_v, out_hbm, c, base)

    return pl.kernel(body,
                     out_type=jax.ShapeDtypeStruct((NC, N_PAD, D),
                                                   jnp.float32),
                     mesh=_mesh(), scratch_types=scratch)


@functools.cache
def _make_sc_agg():
    scratch = [
        pltpu.VMEM((S, CH), jnp.int32),
        pltpu.VMEM((S, CH), jnp.int32),
        pltpu.VMEM((CH, D), jnp.float32),
        pltpu.VMEM((CH, D), jnp.float32),
        pltpu.VMEM_SHARED((N_PAD, D), jnp.float32),
        pltpu.SemaphoreType.DMA,
        pltpu.SemaphoreType.DMA,
    ]

    def body(x_hbm, src_hbm, dst_hbm, zrows_hbm, out_hbm,
             src_v, dst_v, rows0, rows1, acc, sem0, sem1):
        c, s, base = _tile_ids()
        _zero_acc(zrows_hbm, rows0, acc, base)
        plsc.subcore_barrier()

        def run_stage(first_chunk):
            pltpu.sync_copy(src_hbm.at[s, pl.ds(first_chunk, S)], src_v)
            pltpu.sync_copy(dst_hbm.at[s, pl.ds(first_chunk, S)], dst_v)
            pltpu.async_copy(x_hbm.at[src_v.at[0]], rows0, sem0)

            def pair(jj, carry):
                l0 = jj * 2
                pltpu.make_async_copy(x_hbm.at[src_v.at[l0]], rows0,
                                      sem0).wait()
                pltpu.async_copy(x_hbm.at[src_v.at[l0 + 1]], rows1, sem1)
                pltpu.sync_copy(rows0, acc.at[dst_v.at[l0]], add=True)
                pltpu.make_async_copy(x_hbm.at[src_v.at[l0 + 1]], rows1,
                                      sem1).wait()

                @pl.when(l0 + 2 < S)
                def _():
                    pltpu.async_copy(x_hbm.at[src_v.at[l0 + 2]], rows0,
                                     sem0)

                pltpu.sync_copy(rows1, acc.at[dst_v.at[l0 + 1]], add=True)
                return carry

            lax.fori_loop(0, S // 2, pair, 0)

        @pl.when(c == BIG_CORE)
        def _():
            for q in range(K_BIG // S):
                run_stage(q * S)

        @pl.when(c != BIG_CORE)
        def _():
            for q in range((K_ALL - K_BIG) // S):
                run_stage(K_BIG + q * S)

        plsc.subcore_barrier()
        _copy_out(acc, rows0, out_hbm, c, base)

    return pl.kernel(body,
                     out_type=jax.ShapeDtypeStruct((NC, N_PAD, D),
                                                   jnp.float32),
                     mesh=_mesh(), scratch_types=scratch)


_BR = 1024


def _inv_body(c_ref, o_ref):
    o_ref[...] = 1.0 / jnp.maximum(c_ref[0] + c_ref[1], 1.0)


def _inv_counts(cparts):
    return pl.pallas_call(
        _inv_body,
        grid=(N_PAD // _BR,),
        in_specs=[pl.BlockSpec((NC, _BR, D), lambda i: (0, i, 0))],
        out_specs=pl.BlockSpec((_BR, D), lambda i: (i, 0)),
        out_shape=jax.ShapeDtypeStruct((N_PAD, D), jnp.float32),
    )(cparts)


def _dense_body(p_ref, inv_ref, h_ref, wl_ref, bl_ref, wr_ref, o_ref):
    agg = (p_ref[0] + p_ref[1]) * inv_ref[...]
    y = jnp.dot(agg, wl_ref[...], preferred_element_type=jnp.float32)
    y = y + bl_ref[...] + jnp.dot(h_ref[...], wr_ref[...],
                                  preferred_element_type=jnp.float32)
    o_ref[...] = jnp.where(y >= 0, y, 0.01 * y)


def _dense(p, inv, h, wlT, bl2d, wrT):
    return pl.pallas_call(
        _dense_body,
        grid=(N_PAD // _BR,),
        in_specs=[
            pl.BlockSpec((NC, _BR, D), lambda i: (0, i, 0)),
            pl.BlockSpec((_BR, D), lambda i: (i, 0)),
            pl.BlockSpec((_BR, D), lambda i: (i, 0)),
            pl.BlockSpec((D, D), lambda i: (0, 0)),
            pl.BlockSpec((1, D), lambda i: (0, 0)),
            pl.BlockSpec((D, D), lambda i: (0, 0)),
        ],
        out_specs=pl.BlockSpec((_BR, D), lambda i: (i, 0)),
        out_shape=jax.ShapeDtypeStruct((N_PAD, D), jnp.float32),
    )(p, inv, h, wlT, bl2d, wrT)


def kernel(x, edge_index, Wl1, bl1, Wr1, Wl2, bl2, Wr2,
           Wl3, bl3, Wr3, Wl4, bl4, Wr4):
    src = edge_index[0]
    dst = edge_index[1]
    pad = E_PAD - E
    src_p = jnp.concatenate(
        [src, jnp.zeros((pad,), jnp.int32)]).reshape(NS, K_ALL, CH)
    dst_p = jnp.concatenate(
        [dst, jnp.full((pad,), N, jnp.int32)]).reshape(NS, K_ALL, CH)
    zrows = jnp.zeros((CH, D), jnp.float32)
    ones_rows = jnp.ones((CH, D), jnp.float32)

    cparts = _make_sc_counts()(dst_p, zrows, ones_rows)
    inv = _inv_counts(cparts)

    h = jnp.pad(x, ((0, N_PAD - N), (0, 0)))
    for Wl, bl, Wr in ((Wl1, bl1, Wr1), (Wl2, bl2, Wr2),
                       (Wl3, bl3, Wr3), (Wl4, bl4, Wr4)):
        parts = _make_sc_agg()(h, src_p, dst_p, zrows)
        h = _dense(parts, inv, h, Wl.T, bl.reshape(1, D), Wr.T)
    return h[:N]
